# baseline (speedup 1.0000x reference)
import functools

import jax
import jax.numpy as jnp
from jax import lax
from jax.experimental import pallas as pl
from jax.experimental.pallas import tpu as pltpu
from jax.experimental.pallas import tpu_sc as plsc

K = 32
B = 64
N = 32768
D = 64
PK = 8
IPB = 16384
S = (B * N) // IPB
RPB = IPB // PK


def _score_kernel(x_ref, a_ref, bmat_ref, o_ref):
    x = x_ref[...]
    bm = bmat_ref[...]
    lo = jnp.dot(x[:, :256], bm, preferred_element_type=jnp.float32)
    hi = jnp.dot(x[:, 256:], bm, preferred_element_type=jnp.float32)
    proj = jnp.concatenate([lo, hi], axis=1)
    y = proj * x
    row = jax.lax.broadcasted_iota(jnp.int32, (PK, PK * D), 0)
    lane = jax.lax.broadcasted_iota(jnp.int32, (PK, PK * D), 1)
    halves = (lane // D == row).astype(jnp.float32)
    pred = jax.lax.dot_general(halves, y, (((1,), (1,)), ((), ())),
                               preferred_element_type=jnp.float32)
    o_ref[0] = a_ref[0] * pred


def _topk_kernel(s_ref, o_ref):
    def step(i, carry):
        total, consumed = carry
        s = s_ref[...].reshape(B, 2 * PK, RPB)
        m = jnp.max(jnp.max(s, axis=2), axis=1)
        eq = s == m[:, None, None]
        ce = eq.astype(jnp.float32)
        cnt = jnp.sum(jnp.sum(ce, axis=2), axis=1)
        take = jnp.clip(jnp.float32(K) - consumed, 0.0, cnt)
        total = total + jnp.where(take > 0.0, m, 0.0) * take
        consumed = consumed + take
        s_ref[...] = jnp.where(eq, -jnp.inf, s).reshape(S, PK, RPB)
        return total, consumed

    z = jnp.zeros((B,), jnp.float32)
    total, _ = jax.lax.fori_loop(0, K, step, (z, z))
    o_ref[...] = total.reshape(B, 1) * (1.0 / K)


NG = 128
GCH = 16
NEG = float("-inf")
SC_STAGE = 99


def _sc_topk_body(s_hbm, o_hbm, V, GM, ACC, CAND, CNT, OUTROW):
    wid = lax.axis_index("s") * 2 + lax.axis_index("c")

    def do_row(j, _):
        r = wid * 2 + j
        pltpu.sync_copy(s_hbm.at[r], V)
        if SC_STAGE < 2:
            OUTROW[...] = V[pl.ds(0, 16)]
            pltpu.sync_copy(OUTROW, o_hbm.at[r])
            return 0

        def p1(g, _):
            base = g * (GCH * 16)
            acc = V[pl.ds(base, 16)]
            for t in range(1, GCH):
                acc = jnp.maximum(acc, V[pl.ds(base + t * 16, 16)])
            GM[pl.ds(g * 16, 16)] = acc
            return 0
        lax.fori_loop(0, NG, p1, 0)
        if SC_STAGE < 3:
            OUTROW[...] = GM[pl.ds(0, 16)]
            pltpu.sync_copy(OUTROW, o_hbm.at[r])
            return 0

        for a in range(8):
            def p2(i, m, a=a):
                return jnp.maximum(m, GM[pl.ds((i * 8 + a) * 16, 16)])
            ACC[pl.ds(a * 16, 16)] = lax.fori_loop(
                0, NG // 8, p2, jnp.full((16,), NEG, jnp.float32))
        if SC_STAGE < 4:
            OUTROW[...] = ACC[pl.ds(0, 16)]
            pltpu.sync_copy(OUTROW, o_hbm.at[r])
            return 0

        def ext(k, tv):
            m = jnp.full((16,), NEG, jnp.float32)
            for a in range(8):
                m = jnp.maximum(m, ACC[pl.ds(a * 16, 16)])
            msv = jnp.full((16,), jnp.max(m))
            for a in range(8):
                v = ACC[pl.ds(a * 16, 16)]
                ACC[pl.ds(a * 16, 16)] = jnp.where(v == msv, NEG, v)
            return msv
        tv = lax.fori_loop(0, K, ext, jnp.full((16,), NEG, jnp.float32))
        if SC_STAGE < 5:
            OUTROW[...] = tv
            pltpu.sync_copy(OUTROW, o_hbm.at[r])
            return 0

        def p3a(g, _):
            gm = GM[pl.ds(g * 16, 16)]
            CNT[pl.ds(g * 16, 16)] = plsc.all_reduce_population_count(gm >= tv)
            return 0
        lax.fori_loop(0, NG, p3a, 0)
        if SC_STAGE < 6:
            OUTROW[...] = CNT[pl.ds(0, 16)].astype(jnp.float32)
            pltpu.sync_copy(OUTROW, o_hbm.at[r])
            return 0

        def p3b(g, ptr):
            def collect(p):
                for t in range(GCH):
                    c = V[pl.ds(g * (GCH * 16) + t * 16, 16)]
                    msk = c >= tv
                    plsc.store_compressed(CAND.at[pl.ds(p, 16)], c, mask=msk)
                    p = p + jnp.max(plsc.all_reduce_population_count(msk))
                return p
            trig = jnp.max(CNT[pl.ds(g * 16, 16)])
            return lax.cond(trig > 0, collect, lambda p: p, ptr)
        ptr = lax.fori_loop(0, NG, p3b, jnp.int32(0))
        if SC_STAGE < 7:
            OUTROW[...] = jnp.full((16,), ptr).astype(jnp.float32)
            pltpu.sync_copy(OUTROW, o_hbm.at[r])
            return 0

        CAND[pl.ds(ptr, 16)] = jnp.full((16,), NEG, jnp.float32)
        nv = lax.shift_right_logical(ptr + 15, 4)

        def rnd(k, carry):
            total, consumed = carry
            def fmax(i, m):
                return jnp.maximum(m, CAND[pl.ds(i * 16, 16)])
            m = lax.fori_loop(0, nv, fmax, jnp.full((16,), NEG, jnp.float32))
            msv = jnp.full((16,), jnp.max(m))

            def cm(i, pc):
                c = CAND[pl.ds(i * 16, 16)]
                eqm = c == msv
                pc = pc + plsc.all_reduce_population_count(eqm)
                CAND[pl.ds(i * 16, 16)] = jnp.where(eqm, NEG, c)
                return pc
            pc = lax.fori_loop(0, nv, cm, jnp.zeros((16,), jnp.int32))
            cntf = pc.astype(jnp.float32)
            take = jnp.clip(jnp.float32(K) - consumed, 0.0, cntf)
            total = total + jnp.where(take > 0.0, msv, 0.0) * take
            return total, consumed + take

        z = jnp.zeros((16,), jnp.float32)
        total, _ = lax.fori_loop(0, K, rnd, (z, z))
        OUTROW[...] = total * (1.0 / K)
        pltpu.sync_copy(OUTROW, o_hbm.at[r])
        return 0

    lax.fori_loop(0, 2, do_row, 0)


def _sc_topk(scores):
    f = pl.kernel(
        _sc_topk_body,
        out_type=jax.ShapeDtypeStruct((B, 16), jnp.float32),
        mesh=plsc.VectorSubcoreMesh(core_axis_name="c", subcore_axis_name="s"),
        scratch_types=[
            pltpu.VMEM((N,), jnp.float32),
            pltpu.VMEM((NG * 16,), jnp.float32),
            pltpu.VMEM((8 * 16,), jnp.float32),
            pltpu.VMEM((N + 16,), jnp.float32),
            pltpu.VMEM((NG * 16,), jnp.int32),
            pltpu.VMEM((16,), jnp.float32),
        ],
    )
    return f(scores)


@functools.partial(jax.jit, static_argnames=())
def kernel(ins_emb, attn, W):
    d = W.shape[0]
    bq = jnp.zeros((D, D), jnp.float32).at[:d, d:].set(W.T)
    z = jnp.zeros((D, D), jnp.float32)
    bmat = jnp.block([[bq, z, z, z],
                      [z, bq, z, z],
                      [z, z, bq, z],
                      [z, z, z, bq]])
    x8 = ins_emb.reshape(B * N // PK, PK * D)
    attn3 = attn.reshape(S, RPB, PK)
    attn_t = jnp.concatenate(
        [attn3[:, :, p].reshape(S, 1, RPB) for p in range(PK)], axis=1)

    scores = pl.pallas_call(
        _score_kernel,
        grid=(S,),
        in_specs=[
            pl.BlockSpec((RPB, PK * D), lambda s: (s, 0)),
            pl.BlockSpec((1, PK, RPB), lambda s: (s, 0, 0)),
            pl.BlockSpec((4 * D, 4 * D), lambda s: (0, 0)),
        ],
        out_specs=pl.BlockSpec((1, PK, RPB), lambda s: (s, 0, 0)),
        out_shape=jax.ShapeDtypeStruct((S, PK, RPB), jnp.float32),
    )(x8, attn_t, bmat)

    out = pl.pallas_call(
        _topk_kernel,
        grid=(1,),
        in_specs=[pl.BlockSpec((S, PK, RPB), lambda i: (0, 0, 0))],
        out_specs=pl.BlockSpec((B, 1), lambda i: (0, 0)),
        out_shape=jax.ShapeDtypeStruct((B, 1), jnp.float32),
    )(scores)
    return out


# speedup vs baseline: 1.7635x; 1.7635x over previous
import functools

import jax
import jax.numpy as jnp
from jax import lax
from jax.experimental import pallas as pl
from jax.experimental.pallas import tpu as pltpu
from jax.experimental.pallas import tpu_sc as plsc

K = 32
B = 64
N = 32768
D = 64
PK = 8
IPB = 16384
S = (B * N) // IPB
RPB = IPB // PK


def _score_kernel(x_ref, a_ref, bmat_ref, o_ref):
    x = x_ref[...]
    bm = bmat_ref[...]
    lo = jnp.dot(x[:, :256], bm, preferred_element_type=jnp.float32)
    hi = jnp.dot(x[:, 256:], bm, preferred_element_type=jnp.float32)
    proj = jnp.concatenate([lo, hi], axis=1)
    y = proj * x
    row = jax.lax.broadcasted_iota(jnp.int32, (PK, PK * D), 0)
    lane = jax.lax.broadcasted_iota(jnp.int32, (PK, PK * D), 1)
    halves = (lane // D == row).astype(jnp.float32)
    pred = jax.lax.dot_general(halves, y, (((1,), (1,)), ((), ())),
                               preferred_element_type=jnp.float32)
    r8 = jax.lax.broadcasted_iota(jnp.int32, (PK, PK), 0)
    c8 = jax.lax.broadcasted_iota(jnp.int32, (PK, PK), 1)
    eye = (r8 == c8).astype(jnp.float32)
    att = jax.lax.dot_general(eye, a_ref[0], (((1,), (1,)), ((), ())),
                              preferred_element_type=jnp.float32)
    o_ref[0] = att * pred


def _topk_kernel(s_ref, o_ref):
    def step(i, carry):
        total, consumed = carry
        s = s_ref[...].reshape(B, 2 * PK, RPB)
        m = jnp.max(jnp.max(s, axis=2), axis=1)
        eq = s == m[:, None, None]
        ce = eq.astype(jnp.float32)
        cnt = jnp.sum(jnp.sum(ce, axis=2), axis=1)
        take = jnp.clip(jnp.float32(K) - consumed, 0.0, cnt)
        total = total + jnp.where(take > 0.0, m, 0.0) * take
        consumed = consumed + take
        s_ref[...] = jnp.where(eq, -jnp.inf, s).reshape(S, PK, RPB)
        return total, consumed

    z = jnp.zeros((B,), jnp.float32)
    total, _ = jax.lax.fori_loop(0, K, step, (z, z))
    o_ref[...] = total.reshape(B, 1) * (1.0 / K)


NG = 128
GCH = 16
NEG = float("-inf")
SC_STAGE = 99


def _sc_topk_body(s_hbm, o_hbm, V, GM, ACC, CAND, CNT, OUTROW):
    wid = lax.axis_index("s") * 2 + lax.axis_index("c")

    def do_row(j, _):
        r = wid * 2 + j
        pltpu.sync_copy(s_hbm.at[r], V)
        if SC_STAGE < 2:
            OUTROW[...] = V[pl.ds(0, 16)]
            pltpu.sync_copy(OUTROW, o_hbm.at[r])
            return 0

        def p1(g, _):
            base = g * (GCH * 16)
            acc = V[pl.ds(base, 16)]
            for t in range(1, GCH):
                acc = jnp.maximum(acc, V[pl.ds(base + t * 16, 16)])
            GM[pl.ds(g * 16, 16)] = acc
            return 0
        lax.fori_loop(0, NG, p1, 0)
        if SC_STAGE < 3:
            OUTROW[...] = GM[pl.ds(0, 16)]
            pltpu.sync_copy(OUTROW, o_hbm.at[r])
            return 0

        for a in range(8):
            def p2(i, m, a=a):
                return jnp.maximum(m, GM[pl.ds((i * 8 + a) * 16, 16)])
            ACC[pl.ds(a * 16, 16)] = lax.fori_loop(
                0, NG // 8, p2, jnp.full((16,), NEG, jnp.float32))
        if SC_STAGE < 4:
            OUTROW[...] = ACC[pl.ds(0, 16)]
            pltpu.sync_copy(OUTROW, o_hbm.at[r])
            return 0

        def ext(k, tv):
            m = jnp.full((16,), NEG, jnp.float32)
            for a in range(8):
                m = jnp.maximum(m, ACC[pl.ds(a * 16, 16)])
            msv = jnp.full((16,), jnp.max(m))
            for a in range(8):
                v = ACC[pl.ds(a * 16, 16)]
                ACC[pl.ds(a * 16, 16)] = jnp.where(v == msv, NEG, v)
            return msv
        tv = lax.fori_loop(0, K, ext, jnp.full((16,), NEG, jnp.float32))
        if SC_STAGE < 5:
            OUTROW[...] = tv
            pltpu.sync_copy(OUTROW, o_hbm.at[r])
            return 0

        def p3a(g, _):
            gm = GM[pl.ds(g * 16, 16)]
            CNT[pl.ds(g * 16, 16)] = plsc.all_reduce_population_count(gm >= tv)
            return 0
        lax.fori_loop(0, NG, p3a, 0)
        if SC_STAGE < 6:
            OUTROW[...] = CNT[pl.ds(0, 16)].astype(jnp.float32)
            pltpu.sync_copy(OUTROW, o_hbm.at[r])
            return 0

        def p3b(g, ptr):
            def collect(p):
                for t in range(GCH):
                    c = V[pl.ds(g * (GCH * 16) + t * 16, 16)]
                    msk = c >= tv
                    plsc.store_compressed(CAND.at[pl.ds(p, 16)], c, mask=msk)
                    p = p + jnp.max(plsc.all_reduce_population_count(msk))
                return p
            trig = jnp.max(CNT[pl.ds(g * 16, 16)])
            return lax.cond(trig > 0, collect, lambda p: p, ptr)
        ptr = lax.fori_loop(0, NG, p3b, jnp.int32(0))
        if SC_STAGE < 7:
            OUTROW[...] = jnp.full((16,), ptr).astype(jnp.float32)
            pltpu.sync_copy(OUTROW, o_hbm.at[r])
            return 0

        CAND[pl.ds(ptr, 16)] = jnp.full((16,), NEG, jnp.float32)
        nv = lax.shift_right_logical(ptr + 15, 4)

        def rnd(k, carry):
            total, consumed = carry
            def fmax(i, m):
                return jnp.maximum(m, CAND[pl.ds(i * 16, 16)])
            m = lax.fori_loop(0, nv, fmax, jnp.full((16,), NEG, jnp.float32))
            msv = jnp.full((16,), jnp.max(m))

            def cm(i, pc):
                c = CAND[pl.ds(i * 16, 16)]
                eqm = c == msv
                pc = pc + plsc.all_reduce_population_count(eqm)
                CAND[pl.ds(i * 16, 16)] = jnp.where(eqm, NEG, c)
                return pc
            pc = lax.fori_loop(0, nv, cm, jnp.zeros((16,), jnp.int32))
            cntf = pc.astype(jnp.float32)
            take = jnp.clip(jnp.float32(K) - consumed, 0.0, cntf)
            total = total + jnp.where(take > 0.0, msv, 0.0) * take
            return total, consumed + take

        z = jnp.zeros((16,), jnp.float32)
        total, _ = lax.fori_loop(0, K, rnd, (z, z))
        OUTROW[...] = total * (1.0 / K)
        pltpu.sync_copy(OUTROW, o_hbm.at[r])
        return 0

    lax.fori_loop(0, 2, do_row, 0)


def _sc_topk(scores):
    f = pl.kernel(
        _sc_topk_body,
        out_type=jax.ShapeDtypeStruct((B, 16), jnp.float32),
        mesh=plsc.VectorSubcoreMesh(core_axis_name="c", subcore_axis_name="s"),
        scratch_types=[
            pltpu.VMEM((N,), jnp.float32),
            pltpu.VMEM((NG * 16,), jnp.float32),
            pltpu.VMEM((8 * 16,), jnp.float32),
            pltpu.VMEM((N + 16,), jnp.float32),
            pltpu.VMEM((NG * 16,), jnp.int32),
            pltpu.VMEM((16,), jnp.float32),
        ],
    )
    return f(scores)


@functools.partial(jax.jit, static_argnames=())
def kernel(ins_emb, attn, W):
    d = W.shape[0]
    bq = jnp.zeros((D, D), jnp.float32).at[:d, d:].set(W.T)
    z = jnp.zeros((D, D), jnp.float32)
    bmat = jnp.block([[bq, z, z, z],
                      [z, bq, z, z],
                      [z, z, bq, z],
                      [z, z, z, bq]])
    x8 = ins_emb.reshape(B * N // PK, PK * D)
    attn3 = attn.reshape(S, RPB, PK)

    scores = pl.pallas_call(
        _score_kernel,
        grid=(S,),
        in_specs=[
            pl.BlockSpec((RPB, PK * D), lambda s: (s, 0)),
            pl.BlockSpec((1, RPB, PK), lambda s: (s, 0, 0)),
            pl.BlockSpec((4 * D, 4 * D), lambda s: (0, 0)),
        ],
        out_specs=pl.BlockSpec((1, PK, RPB), lambda s: (s, 0, 0)),
        out_shape=jax.ShapeDtypeStruct((S, PK, RPB), jnp.float32),
    )(x8, attn3, bmat)

    out = pl.pallas_call(
        _topk_kernel,
        grid=(1,),
        in_specs=[pl.BlockSpec((S, PK, RPB), lambda i: (0, 0, 0))],
        out_specs=pl.BlockSpec((B, 1), lambda i: (0, 0)),
        out_shape=jax.ShapeDtypeStruct((B, 1), jnp.float32),
    )(scores)
    return out


# speedup vs baseline: 3.3439x; 1.8962x over previous
import functools

import jax
import jax.numpy as jnp
from jax import lax
from jax.experimental import pallas as pl
from jax.experimental.pallas import tpu as pltpu
from jax.experimental.pallas import tpu_sc as plsc

K = 32
B = 64
N = 32768
D = 64
PK = 8
IPB = 16384
S = (B * N) // IPB
RPB = IPB // PK


def _score_kernel(x_ref, a_ref, bmat_ref, o_ref):
    x = x_ref[0]
    proj = jnp.dot(x, bmat_ref[...], preferred_element_type=jnp.float32)
    y = proj * x
    ones = jnp.ones((1, D), jnp.float32)
    pred1 = jax.lax.dot_general(ones, y, (((1,), (1,)), ((), ())),
                                preferred_element_type=jnp.float32)
    pred = jnp.concatenate(
        [pred1[:, p * RPB:(p + 1) * RPB] for p in range(PK)], axis=0)
    o_ref[0] = a_ref[0] * pred


def _topk_kernel(s_ref, o_ref):
    def step(i, carry):
        total, consumed = carry
        s = s_ref[...].reshape(B, 2 * PK, RPB)
        m = jnp.max(jnp.max(s, axis=2), axis=1)
        eq = s == m[:, None, None]
        ce = eq.astype(jnp.float32)
        cnt = jnp.sum(jnp.sum(ce, axis=2), axis=1)
        take = jnp.clip(jnp.float32(K) - consumed, 0.0, cnt)
        total = total + jnp.where(take > 0.0, m, 0.0) * take
        consumed = consumed + take
        s_ref[...] = jnp.where(eq, -jnp.inf, s).reshape(S, PK, RPB)
        return total, consumed

    z = jnp.zeros((B,), jnp.float32)
    total, _ = jax.lax.fori_loop(0, K, step, (z, z))
    o_ref[...] = total.reshape(B, 1) * (1.0 / K)


NG = 128
GCH = 16
NEG = float("-inf")
SC_STAGE = 99


def _sc_topk_body(s_hbm, o_hbm, V, GM, ACC, CAND, CNT, OUTROW):
    wid = lax.axis_index("s") * 2 + lax.axis_index("c")

    def do_row(j, _):
        r = wid * 2 + j
        pltpu.sync_copy(s_hbm.at[r], V)
        if SC_STAGE < 2:
            OUTROW[...] = V[pl.ds(0, 16)]
            pltpu.sync_copy(OUTROW, o_hbm.at[r])
            return 0

        def p1(g, _):
            base = g * (GCH * 16)
            acc = V[pl.ds(base, 16)]
            for t in range(1, GCH):
                acc = jnp.maximum(acc, V[pl.ds(base + t * 16, 16)])
            GM[pl.ds(g * 16, 16)] = acc
            return 0
        lax.fori_loop(0, NG, p1, 0)
        if SC_STAGE < 3:
            OUTROW[...] = GM[pl.ds(0, 16)]
            pltpu.sync_copy(OUTROW, o_hbm.at[r])
            return 0

        for a in range(8):
            def p2(i, m, a=a):
                return jnp.maximum(m, GM[pl.ds((i * 8 + a) * 16, 16)])
            ACC[pl.ds(a * 16, 16)] = lax.fori_loop(
                0, NG // 8, p2, jnp.full((16,), NEG, jnp.float32))
        if SC_STAGE < 4:
            OUTROW[...] = ACC[pl.ds(0, 16)]
            pltpu.sync_copy(OUTROW, o_hbm.at[r])
            return 0

        def ext(k, tv):
            m = jnp.full((16,), NEG, jnp.float32)
            for a in range(8):
                m = jnp.maximum(m, ACC[pl.ds(a * 16, 16)])
            msv = jnp.full((16,), jnp.max(m))
            for a in range(8):
                v = ACC[pl.ds(a * 16, 16)]
                ACC[pl.ds(a * 16, 16)] = jnp.where(v == msv, NEG, v)
            return msv
        tv = lax.fori_loop(0, K, ext, jnp.full((16,), NEG, jnp.float32))
        if SC_STAGE < 5:
            OUTROW[...] = tv
            pltpu.sync_copy(OUTROW, o_hbm.at[r])
            return 0

        def p3a(g, _):
            gm = GM[pl.ds(g * 16, 16)]
            CNT[pl.ds(g * 16, 16)] = plsc.all_reduce_population_count(gm >= tv)
            return 0
        lax.fori_loop(0, NG, p3a, 0)
        if SC_STAGE < 6:
            OUTROW[...] = CNT[pl.ds(0, 16)].astype(jnp.float32)
            pltpu.sync_copy(OUTROW, o_hbm.at[r])
            return 0

        def p3b(g, ptr):
            def collect(p):
                for t in range(GCH):
                    c = V[pl.ds(g * (GCH * 16) + t * 16, 16)]
                    msk = c >= tv
                    plsc.store_compressed(CAND.at[pl.ds(p, 16)], c, mask=msk)
                    p = p + jnp.max(plsc.all_reduce_population_count(msk))
                return p
            trig = jnp.max(CNT[pl.ds(g * 16, 16)])
            return lax.cond(trig > 0, collect, lambda p: p, ptr)
        ptr = lax.fori_loop(0, NG, p3b, jnp.int32(0))
        if SC_STAGE < 7:
            OUTROW[...] = jnp.full((16,), ptr).astype(jnp.float32)
            pltpu.sync_copy(OUTROW, o_hbm.at[r])
            return 0

        CAND[pl.ds(ptr, 16)] = jnp.full((16,), NEG, jnp.float32)
        nv = lax.shift_right_logical(ptr + 15, 4)

        def rnd(k, carry):
            total, consumed = carry
            def fmax(i, m):
                return jnp.maximum(m, CAND[pl.ds(i * 16, 16)])
            m = lax.fori_loop(0, nv, fmax, jnp.full((16,), NEG, jnp.float32))
            msv = jnp.full((16,), jnp.max(m))

            def cm(i, pc):
                c = CAND[pl.ds(i * 16, 16)]
                eqm = c == msv
                pc = pc + plsc.all_reduce_population_count(eqm)
                CAND[pl.ds(i * 16, 16)] = jnp.where(eqm, NEG, c)
                return pc
            pc = lax.fori_loop(0, nv, cm, jnp.zeros((16,), jnp.int32))
            cntf = pc.astype(jnp.float32)
            take = jnp.clip(jnp.float32(K) - consumed, 0.0, cntf)
            total = total + jnp.where(take > 0.0, msv, 0.0) * take
            return total, consumed + take

        z = jnp.zeros((16,), jnp.float32)
        total, _ = lax.fori_loop(0, K, rnd, (z, z))
        OUTROW[...] = total * (1.0 / K)
        pltpu.sync_copy(OUTROW, o_hbm.at[r])
        return 0

    lax.fori_loop(0, 2, do_row, 0)


def _sc_topk(scores):
    f = pl.kernel(
        _sc_topk_body,
        out_type=jax.ShapeDtypeStruct((B, 16), jnp.float32),
        mesh=plsc.VectorSubcoreMesh(core_axis_name="c", subcore_axis_name="s"),
        scratch_types=[
            pltpu.VMEM((N,), jnp.float32),
            pltpu.VMEM((NG * 16,), jnp.float32),
            pltpu.VMEM((8 * 16,), jnp.float32),
            pltpu.VMEM((N + 16,), jnp.float32),
            pltpu.VMEM((NG * 16,), jnp.int32),
            pltpu.VMEM((16,), jnp.float32),
        ],
    )
    return f(scores)


@functools.partial(jax.jit, static_argnames=())
def kernel(ins_emb, attn, W):
    d = W.shape[0]
    bmat = jnp.zeros((D, D), jnp.float32).at[:d, d:].set(W.T)
    x8 = ins_emb.reshape(S, IPB, D)
    attn3 = attn.reshape(S, PK, RPB)

    scores = pl.pallas_call(
        _score_kernel,
        grid=(S,),
        in_specs=[
            pl.BlockSpec((1, IPB, D), lambda s: (s, 0, 0)),
            pl.BlockSpec((1, PK, RPB), lambda s: (s, 0, 0)),
            pl.BlockSpec((D, D), lambda s: (0, 0)),
        ],
        out_specs=pl.BlockSpec((1, PK, RPB), lambda s: (s, 0, 0)),
        out_shape=jax.ShapeDtypeStruct((S, PK, RPB), jnp.float32),
    )(x8, attn3, bmat)

    out = pl.pallas_call(
        _topk_kernel,
        grid=(1,),
        in_specs=[pl.BlockSpec((S, PK, RPB), lambda i: (0, 0, 0))],
        out_specs=pl.BlockSpec((B, 1), lambda i: (0, 0)),
        out_shape=jax.ShapeDtypeStruct((B, 1), jnp.float32),
    )(scores)
    return out
